# baseline (speedup 1.0000x reference)
import jax
import jax.numpy as jnp
from jax.experimental import pallas as pl
from jax.experimental.pallas import tpu as pltpu

_N = 10000
_D = 128
_BLK = 1000


def _mm_body(agg_ref, nd_ref, W_ref, b_ref, W2_ref, b2_ref, out_ref):
    h = agg_ref[...] * nd_ref[...]
    h = jnp.dot(h, W_ref[...], preferred_element_type=jnp.float32) + b_ref[...]
    out_ref[...] = (
        jnp.dot(h, W2_ref[...], preferred_element_type=jnp.float32) + b2_ref[...]
    )


def kernel(x, edge_index, W, b, W2, b2):
    src = edge_index[0]
    dst = edge_index[1]
    deg_out = jnp.zeros((_N,), dtype=jnp.float32).at[src].add(1.0)
    deg_in = jnp.zeros((_N,), dtype=jnp.float32).at[dst].add(1.0)
    norm_src = jax.lax.rsqrt(jnp.clip(deg_out, 1.0, None))
    norm_dst = jax.lax.rsqrt(jnp.clip(deg_in, 1.0, None))
    h = x * norm_src[:, None]
    msg = h[src]
    agg = jax.ops.segment_sum(msg, dst, num_segments=_N)

    out = pl.pallas_call(
        _mm_body,
        grid=(_N // _BLK,),
        in_specs=[
            pl.BlockSpec((_BLK, _D), lambda i: (i, 0)),
            pl.BlockSpec((_BLK, 1), lambda i: (i, 0)),
            pl.BlockSpec((_D, _D), lambda i: (0, 0)),
            pl.BlockSpec((1, _D), lambda i: (0, 0)),
            pl.BlockSpec((_D, _D), lambda i: (0, 0)),
            pl.BlockSpec((1, _D), lambda i: (0, 0)),
        ],
        out_specs=pl.BlockSpec((_BLK, _D), lambda i: (i, 0)),
        out_shape=jax.ShapeDtypeStruct((_N, _D), jnp.float32),
    )(agg, norm_dst[:, None], W, b[None, :], W2, b2[None, :])
    return out


# speedup vs baseline: 3.7233x; 3.7233x over previous
import functools

import jax
import jax.numpy as jnp
from jax import lax
from jax.experimental import pallas as pl
from jax.experimental.pallas import tpu as pltpu
from jax.experimental.pallas import tpu_sc as plsc

_N = 10000
_E = 320000
_D = 128

_NC = 2
_NS = 16
_L = 16
_NW = _NC * _NS

_NP = 10240

_E_W = _E // _NW

_ROWS_T = _NP // _NS
_C = 128
_CHUNKS = 2560
_CHUNKS_W = _CHUNKS // _NW
_EP = _CHUNKS * _C
_BB = 40

_mesh = plsc.VectorSubcoreMesh(core_axis_name="c", subcore_axis_name="s")


@functools.partial(
    pl.kernel,
    out_type=[
        jax.ShapeDtypeStruct((_NW * _NP,), jnp.float32),
        jax.ShapeDtypeStruct((_NW * _NP,), jnp.float32),
    ],
    mesh=_mesh,
    scratch_types=[
        pltpu.VMEM((1, _E_W), jnp.int32),
        pltpu.VMEM((1, _E_W), jnp.int32),
        pltpu.VMEM((_N,), jnp.float32),
        pltpu.VMEM((_N,), jnp.float32),
    ],
    compiler_params=pltpu.CompilerParams(needs_layout_passes=False),
)
def _sc_degrees(src_hbm, dst_hbm, hs_out, hd_out, sidx, didx, hist_s, hist_d):
    w = lax.axis_index("s") * _NC + lax.axis_index("c")
    pltpu.sync_copy(src_hbm.at[w], sidx)
    pltpu.sync_copy(dst_hbm.at[w], didx)

    zeros = jnp.zeros((_L,), jnp.float32)

    def _zero(i, carry):
        hist_s[pl.ds(i * _L, _L)] = zeros
        hist_d[pl.ds(i * _L, _L)] = zeros
        return carry

    lax.fori_loop(0, _N // _L, _zero, 0)

    ones = jnp.full((_L,), 1.0, jnp.float32)

    def _acc(i, carry):
        s = sidx[0, pl.ds(i * _L, _L)]
        d = didx[0, pl.ds(i * _L, _L)]
        plsc.addupdate_scatter(hist_s, [s], ones)
        plsc.addupdate_scatter(hist_d, [d], ones)
        return carry

    lax.fori_loop(0, _E_W // _L, _acc, 0)

    pltpu.sync_copy(hist_s, hs_out.at[pl.ds(w * _NP, _N)])
    pltpu.sync_copy(hist_d, hd_out.at[pl.ds(w * _NP, _N)])


def _tc_norms_body(hs_ref, hd_ref, ns_ref, nd_ref):
    deg_s = jnp.sum(hs_ref[...], axis=0, keepdims=True)
    deg_d = jnp.sum(hd_ref[...], axis=0, keepdims=True)
    ns_ref[...] = lax.rsqrt(jnp.maximum(deg_s, 1.0))
    nd_ref[...] = lax.rsqrt(jnp.maximum(deg_d, 1.0))


def _tc_norms(hist_s, hist_d):
    return pl.pallas_call(
        _tc_norms_body,
        out_shape=[
            jax.ShapeDtypeStruct((1, _NP), jnp.float32),
            jax.ShapeDtypeStruct((1, _NP), jnp.float32),
        ],
    )(hist_s, hist_d)


_BLK = 640


def _tc_scale_mm_body(ns_ref, x_ref, W_ref, W2_ref, y_ref):
    Wc = jnp.dot(W_ref[...], W2_ref[...], preferred_element_type=jnp.float32)
    y_ref[...] = (
        jnp.dot(x_ref[...], Wc, preferred_element_type=jnp.float32) * ns_ref[...]
    )


def _tc_scale_mm(norm_src, x, W, W2):
    return pl.pallas_call(
        _tc_scale_mm_body,
        grid=(_NP // _BLK,),
        in_specs=[
            pl.BlockSpec((_BLK, 1), lambda i: (i, 0)),
            pl.BlockSpec((_BLK, _D), lambda i: (i, 0)),
            pl.BlockSpec((_D, _D), lambda i: (0, 0)),
            pl.BlockSpec((_D, _D), lambda i: (0, 0)),
        ],
        out_specs=pl.BlockSpec((_BLK, _D), lambda i: (i, 0)),
        out_shape=jax.ShapeDtypeStruct((_NP, _D), jnp.float32),
    )(norm_src, x, W, W2)


@functools.partial(
    pl.kernel,
    out_type=jax.ShapeDtypeStruct((_NC, _NP, _D), jnp.float32),
    mesh=_mesh,
    scratch_types=[
        pltpu.VMEM((_BB, 1, _C), jnp.int32),
        pltpu.VMEM((_BB, 1, _C), jnp.int32),
        pltpu.VMEM((_C, _D), jnp.float32),
        pltpu.VMEM_SHARED((_NP, _D), jnp.float32),
        pltpu.SemaphoreType.DMA,
    ],
    compiler_params=pltpu.CompilerParams(needs_layout_passes=False),
)
def _sc_aggregate(y_hbm, src3_hbm, dst3_hbm, zero_hbm, agg_hbm,
                  sidx, didx, rows, acc, sem):
    c = lax.axis_index("c")
    s = lax.axis_index("s")
    r0 = s * _ROWS_T

    pltpu.sync_copy(zero_hbm.at[pl.ds(r0, _ROWS_T)], acc.at[pl.ds(r0, _ROWS_T)])

    plsc.subcore_barrier()

    w = s * _NC + c
    k0 = w * _CHUNKS_W

    def _edge_batch(m, carry):
        pltpu.sync_copy(src3_hbm.at[pl.ds(k0 + m * _BB, _BB)], sidx)
        pltpu.sync_copy(dst3_hbm.at[pl.ds(k0 + m * _BB, _BB)], didx)

        def _edge_chunk(j, carry2):
            pltpu.async_copy(y_hbm.at[sidx.at[j, 0]], rows, sem).wait()
            pltpu.sync_copy(rows, acc.at[didx.at[j, 0]], add=True)
            return carry2

        return lax.fori_loop(0, _BB, _edge_chunk, carry)

    lax.fori_loop(0, _CHUNKS_W // _BB, _edge_batch, 0)

    plsc.subcore_barrier()

    pltpu.sync_copy(
        acc.at[pl.ds(r0, _ROWS_T)], agg_hbm.at[c, pl.ds(r0, _ROWS_T)]
    )


_BLK4 = 1000


def _tc_out_body(agg_ref, nd_ref, b_ref, W2_ref, b2_ref, out_ref):
    bc = (
        jnp.dot(b_ref[...], W2_ref[...], preferred_element_type=jnp.float32)
        + b2_ref[...]
    )
    a = agg_ref[0] + agg_ref[1]
    out_ref[...] = a * nd_ref[...] + bc


def _tc_out(agg, norm_dst, b, W2, b2):
    return pl.pallas_call(
        _tc_out_body,
        grid=(_N // _BLK4,),
        in_specs=[
            pl.BlockSpec((_NC, _BLK4, _D), lambda i: (0, i, 0)),
            pl.BlockSpec((_BLK4, 1), lambda i: (i, 0)),
            pl.BlockSpec((1, _D), lambda i: (0, 0)),
            pl.BlockSpec((_D, _D), lambda i: (0, 0)),
            pl.BlockSpec((1, _D), lambda i: (0, 0)),
        ],
        out_specs=pl.BlockSpec((_BLK4, _D), lambda i: (i, 0)),
        out_shape=jax.ShapeDtypeStruct((_N, _D), jnp.float32),
    )(agg, norm_dst, b[None, :], W2, b2[None, :])


def kernel(x, edge_index, W, b, W2, b2):
    src = edge_index[0]
    dst = edge_index[1]

    hs_flat, hd_flat = _sc_degrees(
        src.reshape(_NW, 1, _E_W), dst.reshape(_NW, 1, _E_W)
    )
    ns_flat, nd_flat = _tc_norms(
        hs_flat.reshape(_NW, _NP), hd_flat.reshape(_NW, _NP)
    )
    norm_src = ns_flat[0, :_N, None]
    norm_dst = nd_flat[0, :_N, None]

    y = _tc_scale_mm(norm_src, x, W, W2)

    pad = _EP - _E
    src3 = jnp.concatenate([src, jnp.zeros((pad,), jnp.int32)]).reshape(
        _CHUNKS, 1, _C
    )
    dst3 = jnp.concatenate([dst, jnp.full((pad,), _N, jnp.int32)]).reshape(
        _CHUNKS, 1, _C
    )
    zeros = jnp.zeros((_NP, _D), jnp.float32)
    agg = _sc_aggregate(y, src3, dst3, zeros)

    return _tc_out(agg, norm_dst, b, W2, b2)


# speedup vs baseline: 4.0596x; 1.0903x over previous
import functools

import jax
import jax.numpy as jnp
from jax import lax
from jax.experimental import pallas as pl
from jax.experimental.pallas import tpu as pltpu
from jax.experimental.pallas import tpu_sc as plsc

_N = 10000
_E = 320000
_D = 128

_NC = 2
_NS = 16
_L = 16
_NW = _NC * _NS

_NP = 10240

_E_W = _E // _NW

_ROWS_T = _NP // _NS
_C = 128
_CHUNKS = 2560
_CHUNKS_W = _CHUNKS // _NW
_EP = _CHUNKS * _C
_BB = 40

_mesh = plsc.VectorSubcoreMesh(core_axis_name="c", subcore_axis_name="s")


@functools.partial(
    pl.kernel,
    out_type=[
        jax.ShapeDtypeStruct((_NW * _NP,), jnp.float32),
        jax.ShapeDtypeStruct((_NW * _NP,), jnp.float32),
    ],
    mesh=_mesh,
    scratch_types=[
        pltpu.VMEM((1, _E_W), jnp.int32),
        pltpu.VMEM((1, _E_W), jnp.int32),
        pltpu.VMEM((_N,), jnp.float32),
        pltpu.VMEM((_N,), jnp.float32),
    ],
    compiler_params=pltpu.CompilerParams(needs_layout_passes=False),
)
def _sc_degrees(src_hbm, dst_hbm, hs_out, hd_out, sidx, didx, hist_s, hist_d):
    w = lax.axis_index("s") * _NC + lax.axis_index("c")
    pltpu.sync_copy(src_hbm.at[w], sidx)
    pltpu.sync_copy(dst_hbm.at[w], didx)

    zeros = jnp.zeros((_L,), jnp.float32)

    def _zero(i, carry):
        hist_s[pl.ds(i * _L, _L)] = zeros
        hist_d[pl.ds(i * _L, _L)] = zeros
        return carry

    lax.fori_loop(0, _N // _L, _zero, 0)

    ones = jnp.full((_L,), 1.0, jnp.float32)

    def _acc(i, carry):
        s = sidx[0, pl.ds(i * _L, _L)]
        d = didx[0, pl.ds(i * _L, _L)]
        plsc.addupdate_scatter(hist_s, [s], ones)
        plsc.addupdate_scatter(hist_d, [d], ones)
        return carry

    lax.fori_loop(0, _E_W // _L, _acc, 0)

    pltpu.sync_copy(hist_s, hs_out.at[pl.ds(w * _NP, _N)])
    pltpu.sync_copy(hist_d, hd_out.at[pl.ds(w * _NP, _N)])


def _tc_norms_body(hs_ref, hd_ref, ns_ref, nd_ref):
    deg_s = jnp.sum(hs_ref[...], axis=0, keepdims=True)
    deg_d = jnp.sum(hd_ref[...], axis=0, keepdims=True)
    ns_ref[...] = lax.rsqrt(jnp.maximum(deg_s, 1.0))
    nd_ref[...] = lax.rsqrt(jnp.maximum(deg_d, 1.0))


def _tc_norms(hist_s, hist_d):
    return pl.pallas_call(
        _tc_norms_body,
        out_shape=[
            jax.ShapeDtypeStruct((1, _NP), jnp.float32),
            jax.ShapeDtypeStruct((1, _NP), jnp.float32),
        ],
    )(hist_s, hist_d)


_BLK = 640


def _tc_scale_mm_body(ns_ref, x_ref, W_ref, W2_ref, y_ref):
    Wc = jnp.dot(W_ref[...], W2_ref[...], preferred_element_type=jnp.float32)
    y_ref[...] = (
        jnp.dot(x_ref[...], Wc, preferred_element_type=jnp.float32) * ns_ref[...]
    )


def _tc_scale_mm(norm_src, x, W, W2):
    return pl.pallas_call(
        _tc_scale_mm_body,
        grid=(_NP // _BLK,),
        in_specs=[
            pl.BlockSpec((_BLK, 1), lambda i: (i, 0)),
            pl.BlockSpec((_BLK, _D), lambda i: (i, 0)),
            pl.BlockSpec((_D, _D), lambda i: (0, 0)),
            pl.BlockSpec((_D, _D), lambda i: (0, 0)),
        ],
        out_specs=pl.BlockSpec((_BLK, _D), lambda i: (i, 0)),
        out_shape=jax.ShapeDtypeStruct((_NP, _D), jnp.float32),
    )(norm_src, x, W, W2)


@functools.partial(
    pl.kernel,
    out_type=jax.ShapeDtypeStruct((_NC, _NP, _D), jnp.float32),
    mesh=_mesh,
    scratch_types=[
        pltpu.VMEM((_BB, 1, _C), jnp.int32),
        pltpu.VMEM((_BB, 1, _C), jnp.int32),
        pltpu.VMEM((_C, _D), jnp.float32),
        pltpu.VMEM((_C, _D), jnp.float32),
        pltpu.VMEM_SHARED((_NP, _D), jnp.float32),
        pltpu.SemaphoreType.DMA,
        pltpu.SemaphoreType.DMA,
    ],
    compiler_params=pltpu.CompilerParams(needs_layout_passes=False),
)
def _sc_aggregate(y_hbm, src3_hbm, dst3_hbm, zero_hbm, agg_hbm,
                  sidx, didx, rows0, rows1, acc, gsem0, gsem1):
    c = lax.axis_index("c")
    s = lax.axis_index("s")
    r0 = s * _ROWS_T

    pltpu.sync_copy(zero_hbm.at[pl.ds(r0, _ROWS_T)], acc.at[pl.ds(r0, _ROWS_T)])

    plsc.subcore_barrier()

    w = s * _NC + c
    k0 = w * _CHUNKS_W
    n_pairs = _BB // 2

    def _edge_batch(m, carry):
        pltpu.sync_copy(src3_hbm.at[pl.ds(k0 + m * _BB, _BB)], sidx)
        pltpu.sync_copy(dst3_hbm.at[pl.ds(k0 + m * _BB, _BB)], didx)

        pltpu.async_copy(y_hbm.at[sidx.at[0, 0]], rows0, gsem0)

        def _pair(p, carry2):
            j0 = 2 * p
            cp1 = pltpu.async_copy(y_hbm.at[sidx.at[j0 + 1, 0]], rows1, gsem1)
            pltpu.make_async_copy(y_hbm.at[sidx.at[j0, 0]], rows0, gsem0).wait()
            pltpu.sync_copy(rows0, acc.at[didx.at[j0, 0]], add=True)

            @pl.when(p < n_pairs - 1)
            def _():
                pltpu.async_copy(y_hbm.at[sidx.at[j0 + 2, 0]], rows0, gsem0)

            cp1.wait()
            pltpu.sync_copy(rows1, acc.at[didx.at[j0 + 1, 0]], add=True)
            return carry2

        return lax.fori_loop(0, n_pairs, _pair, carry)

    lax.fori_loop(0, _CHUNKS_W // _BB, _edge_batch, 0)

    plsc.subcore_barrier()

    pltpu.sync_copy(
        acc.at[pl.ds(r0, _ROWS_T)], agg_hbm.at[c, pl.ds(r0, _ROWS_T)]
    )


_BLK4 = 1000


def _tc_out_body(agg_ref, nd_ref, b_ref, W2_ref, b2_ref, out_ref):
    bc = (
        jnp.dot(b_ref[...], W2_ref[...], preferred_element_type=jnp.float32)
        + b2_ref[...]
    )
    a = agg_ref[0] + agg_ref[1]
    out_ref[...] = a * nd_ref[...] + bc


def _tc_out(agg, norm_dst, b, W2, b2):
    return pl.pallas_call(
        _tc_out_body,
        grid=(_N // _BLK4,),
        in_specs=[
            pl.BlockSpec((_NC, _BLK4, _D), lambda i: (0, i, 0)),
            pl.BlockSpec((_BLK4, 1), lambda i: (i, 0)),
            pl.BlockSpec((1, _D), lambda i: (0, 0)),
            pl.BlockSpec((_D, _D), lambda i: (0, 0)),
            pl.BlockSpec((1, _D), lambda i: (0, 0)),
        ],
        out_specs=pl.BlockSpec((_BLK4, _D), lambda i: (i, 0)),
        out_shape=jax.ShapeDtypeStruct((_N, _D), jnp.float32),
    )(agg, norm_dst, b[None, :], W2, b2[None, :])


def kernel(x, edge_index, W, b, W2, b2):
    src = edge_index[0]
    dst = edge_index[1]

    hs_flat, hd_flat = _sc_degrees(
        src.reshape(_NW, 1, _E_W), dst.reshape(_NW, 1, _E_W)
    )
    ns_flat, nd_flat = _tc_norms(
        hs_flat.reshape(_NW, _NP), hd_flat.reshape(_NW, _NP)
    )
    norm_src = ns_flat[0, :_N, None]
    norm_dst = nd_flat[0, :_N, None]

    y = _tc_scale_mm(norm_src, x, W, W2)

    pad = _EP - _E
    src3 = jnp.concatenate([src, jnp.zeros((pad,), jnp.int32)]).reshape(
        _CHUNKS, 1, _C
    )
    dst3 = jnp.concatenate([dst, jnp.full((pad,), _N, jnp.int32)]).reshape(
        _CHUNKS, 1, _C
    )
    zeros = jnp.zeros((_NP, _D), jnp.float32)
    agg = _sc_aggregate(y, src3, dst3, zeros)

    return _tc_out(agg, norm_dst, b, W2, b2)
